# baseline (speedup 1.0000x reference)
import functools

import jax
import jax.numpy as jnp
from jax import lax
from jax.experimental import pallas as pl
from jax.experimental.pallas import tpu as pltpu
from jax.experimental.pallas import tpu_sc as plsc

D = 128
SCALE = 1.0 / (3 * D)

_NC = 2
_NS = 16
_NW = _NC * _NS
_LANES = 16

_CHUNK = 8192


def _rowsum_body(t_ref, o_ref):
    ones = jnp.ones((D, 1), jnp.float32)
    o_ref[...] = lax.dot(t_ref[...], ones, precision=lax.Precision.DEFAULT,
                         preferred_element_type=jnp.float32)


def _row_sums_padded(table, blk=8192):
    v = table.shape[0]
    g = pl.cdiv(v, blk)
    out = pl.pallas_call(
        _rowsum_body,
        grid=(g,),
        in_specs=[pl.BlockSpec((blk, D), lambda i: (i, 0))],
        out_specs=pl.BlockSpec((blk, 1), lambda i: (i, 0)),
        out_shape=jax.ShapeDtypeStruct((g * blk, 1), jnp.float32),
    )(table)
    return out.reshape(-1)


def _tail_body(t_ref, s_ref, o_ref):
    del s_ref
    _rowsum_body(t_ref, o_ref)


def _row_sums_tail(table, s_main, blk_idx, blk=8192):
    n = s_main.shape[0]
    s_main = s_main.reshape(n, 1)
    out = pl.pallas_call(
        _tail_body,
        grid=(1,),
        in_specs=[
            pl.BlockSpec((blk, D), lambda i: (blk_idx, 0)),
            pl.BlockSpec((blk, 1), lambda i: (blk_idx, 0)),
        ],
        out_specs=pl.BlockSpec((blk, 1), lambda i: (blk_idx, 0)),
        out_shape=jax.ShapeDtypeStruct((n, 1), jnp.float32),
        input_output_aliases={1: 0},
    )(table, s_main)
    return out.reshape(-1)


_RC = 256
_NCH_RS = 30
_SC_ROWS = _NW * _NCH_RS * _RC
_SC_OUT_PAD = _SC_ROWS + 5120


def _sc_row_sums(table):
    mesh = plsc.VectorSubcoreMesh(core_axis_name="c", subcore_axis_name="s")

    @functools.partial(
        pl.kernel,
        mesh=mesh,
        compiler_params=pltpu.CompilerParams(needs_layout_passes=False),
        out_type=jax.ShapeDtypeStruct((_SC_OUT_PAD,), jnp.float32),
        scratch_types=[
            pltpu.VMEM((_RC, D), jnp.float32),
            pltpu.VMEM((_RC, D), jnp.float32),
            pltpu.VMEM((_RC,), jnp.float32),
            pltpu.VMEM((_RC,), jnp.float32),
            pltpu.SemaphoreType.DMA,
            pltpu.SemaphoreType.DMA,
            pltpu.SemaphoreType.DMA,
            pltpu.SemaphoreType.DMA,
        ],
    )
    def k(tab_h, out_h, r0, r1, o0, o1, si0, si1, so0, so1):
        wid = lax.axis_index("s") * _NC + lax.axis_index("c")
        row0 = wid * _NCH_RS * _RC
        lanes = lax.iota(jnp.int32, _LANES)
        idx15 = lanes * 0 + (_LANES - 1)
        rbufs = (r0, r1)
        obufs = (o0, o1)
        sins = (si0, si1)
        souts = (so0, so1)

        def fire(c, p):
            return pltpu.async_copy(tab_h.at[pl.ds(row0 + c * _RC, _RC)],
                                    rbufs[p], sins[p])

        def wait_in(c, p):
            pltpu.make_async_copy(tab_h.at[pl.ds(row0 + c * _RC, _RC)],
                                  rbufs[p], sins[p]).wait()

        def drain_out(p):
            pltpu.make_async_copy(obufs[p], out_h.at[pl.ds(row0, _RC)],
                                  souts[p]).wait()

        def process(c, p):
            rbuf = rbufs[p]
            obuf = obufs[p]

            def group(g, carry):
                acc = jnp.zeros((_LANES,), jnp.float32)
                for j in range(_LANES):
                    r = g * _LANES + j
                    v = rbuf[r, pl.ds(0, _LANES)]
                    for kk in range(1, D // _LANES):
                        v = v + rbuf[r, pl.ds(kk * _LANES, _LANES)]
                    cum = plsc.cumsum(v)
                    tot = lax.gather(
                        cum, idx15.reshape(_LANES, 1),
                        dimension_numbers=lax.GatherDimensionNumbers(
                            offset_dims=(), collapsed_slice_dims=(0,),
                            start_index_map=(0,)),
                        slice_sizes=(1,),
                        mode=lax.GatherScatterMode.PROMISE_IN_BOUNDS)
                    acc = jnp.where(lanes == j, tot, acc)
                obuf[pl.ds(g * _LANES, _LANES)] = acc
                return carry

            lax.fori_loop(0, _RC // _LANES, group, 0)
            pltpu.async_copy(obuf, out_h.at[pl.ds(row0 + c * _RC, _RC)],
                             souts[p])

        fire(0, 0)
        fire(1, 1)

        def step(i, carry):
            for par in (0, 1):
                c = 2 * i + par
                wait_in(c, par)

                @pl.when(c >= 2)
                def _():
                    drain_out(par)

                process(c, par)

                @pl.when(c + 2 < _NCH_RS)
                def _():
                    fire(c + 2, par)

            return carry

        lax.fori_loop(0, _NCH_RS // 2, step, 0)
        drain_out(0)
        drain_out(1)

    return k(table)


def _gather_one(idx, tab):
    bl = idx.shape[0]
    tab_n = tab.shape[0]
    per_w = bl // _NW
    nch = per_w // _CHUNK
    mesh = plsc.VectorSubcoreMesh(core_axis_name="c", subcore_axis_name="s")

    @functools.partial(
        pl.kernel,
        mesh=mesh,
        out_type=jax.ShapeDtypeStruct((bl,), jnp.float32),
        scratch_types=[
            pltpu.VMEM((_CHUNK,), jnp.int32),
            pltpu.VMEM((_CHUNK,), jnp.int32),
            pltpu.VMEM((_CHUNK,), jnp.float32),
            pltpu.VMEM((_CHUNK,), jnp.float32),
            pltpu.VMEM_SHARED((tab_n,), jnp.float32),
            pltpu.SemaphoreType.DMA,
            pltpu.SemaphoreType.DMA,
        ],
    )
    def k(idx_h, tab_h, out_h, i0, i1, v0, v1, tab_sh, sem0, sem1):
        wid = lax.axis_index("s") * _NC + lax.axis_index("c")
        idx_bufs = (i0, i1)
        val_bufs = (v0, v1)
        sems = (sem0, sem1)

        def load_idx(c, p):
            base = wid * per_w + c * _CHUNK
            pltpu.sync_copy(idx_h.at[pl.ds(base, _CHUNK)], idx_bufs[p])

        def fire(p):
            return pltpu.async_copy(tab_sh.at[idx_bufs[p]], val_bufs[p],
                                    sems[p])

        pending = {}
        load_idx(0, 0)

        @pl.when(lax.axis_index("s") == 0)
        def _stage():
            pltpu.sync_copy(tab_h, tab_sh)

        plsc.subcore_barrier()
        pending[0] = fire(0)
        for c in range(nch):
            p = c % 2
            if c + 1 < nch:
                load_idx(c + 1, 1 - p)
                pending[c + 1] = fire(1 - p)
            pending.pop(c).wait()
            base = wid * per_w + c * _CHUNK
            pltpu.sync_copy(val_bufs[p], out_h.at[pl.ds(base, _CHUNK)])

    return k(idx, tab)


def _mlp_body(xw_ref, x2_ref, x3_ref, w1_ref, b1_ref, w2_ref, b2_ref, o_ref):
    x = xw_ref[...] + x2_ref[...] + x3_ref[...]
    w1 = w1_ref[...] * SCALE
    h = lax.dot(x, w1, precision=lax.Precision.DEFAULT,
                preferred_element_type=jnp.float32)
    h = jnp.maximum(h + b1_ref[...], 0.0)
    o_ref[...] = lax.dot(h, w2_ref[...], precision=lax.Precision.DEFAULT,
                         preferred_element_type=jnp.float32) + b2_ref[...]


def _mlp(xw, x2, x3, w1, b1, w2, b2, blk_b=1024):
    b, l = xw.shape
    n = w2.shape[1]
    xspec = pl.BlockSpec((blk_b, l), lambda i: (i, 0))
    return pl.pallas_call(
        _mlp_body,
        grid=(b // blk_b,),
        in_specs=[
            xspec,
            xspec,
            xspec,
            pl.BlockSpec((l, D), lambda i: (0, 0)),
            pl.BlockSpec((1, D), lambda i: (0, 0)),
            pl.BlockSpec((D, n), lambda i: (0, 0)),
            pl.BlockSpec((1, n), lambda i: (0, 0)),
        ],
        out_specs=pl.BlockSpec((blk_b, n), lambda i: (i, 0)),
        out_shape=jax.ShapeDtypeStruct((b, n), jnp.float32),
    )(xw, x2, x3, w1, b1, w2, b2)


def kernel(input_ids, input_ids_gram2, input_ids_gram3, input_mask, labels,
           emb_word, emb_g2, emb_g3, W1, b1, W2, b2):
    b, l = input_ids.shape
    v3 = emb_g3.shape[0]
    del v3
    s3_main = _sc_row_sums(emb_g3)
    sw = _row_sums_padded(emb_word)
    xw = _gather_one(input_ids.reshape(-1), sw)
    s2 = _row_sums_padded(emb_g2)
    x2 = _gather_one(input_ids_gram2.reshape(-1), s2)
    s3 = _row_sums_tail(emb_g3, s3_main, _SC_ROWS // 8192)
    x3 = _gather_one(input_ids_gram3.reshape(-1), s3)
    return _mlp(xw.reshape(b, l), x2.reshape(b, l), x3.reshape(b, l),
                W1, b1.reshape(1, -1), W2, b2.reshape(1, -1))


# speedup vs baseline: 1.3666x; 1.3666x over previous
import functools

import jax
import jax.numpy as jnp
from jax import lax
from jax.experimental import pallas as pl
from jax.experimental.pallas import tpu as pltpu
from jax.experimental.pallas import tpu_sc as plsc

D = 128
SCALE = 1.0 / (3 * D)

_NC = 2
_NS = 16
_NW = _NC * _NS
_LANES = 16

_CHUNK = 8192


def _rowsum_body(t_ref, o_ref):
    ones = jnp.ones((D, 1), jnp.float32)
    o_ref[...] = lax.dot(t_ref[...], ones, precision=lax.Precision.DEFAULT,
                         preferred_element_type=jnp.float32)


def _row_sums_padded(table, blk=8192, nblocks=None):
    v = table.shape[0]
    g = pl.cdiv(v, blk) if nblocks is None else nblocks
    out = pl.pallas_call(
        _rowsum_body,
        grid=(g,),
        in_specs=[pl.BlockSpec((blk, D), lambda i: (i, 0))],
        out_specs=pl.BlockSpec((blk, 1), lambda i: (i, 0)),
        out_shape=jax.ShapeDtypeStruct((g * blk, 1), jnp.float32),
    )(table)
    return out.reshape(-1)


def _row_sums_block(table, blk_idx, blk=8192):
    return pl.pallas_call(
        _rowsum_body,
        grid=(1,),
        in_specs=[pl.BlockSpec((blk, D), lambda i: (blk_idx, 0))],
        out_specs=pl.BlockSpec((blk, 1), lambda i: (0, 0)),
        out_shape=jax.ShapeDtypeStruct((blk, 1), jnp.float32),
    )(table).reshape(-1)


_RC = 256
_NCH_RS = 30
_SC_ROWS = _NW * _NCH_RS * _RC
_G2_SPLIT = _SC_ROWS - _NW * 8 * _RC


def _sc_row_sums(table, row_off, nch):
    mesh = plsc.VectorSubcoreMesh(core_axis_name="c", subcore_axis_name="s")

    @functools.partial(
        pl.kernel,
        mesh=mesh,
        compiler_params=pltpu.CompilerParams(needs_layout_passes=False),
        out_type=jax.ShapeDtypeStruct((_NW * nch * _RC,), jnp.float32),
        scratch_types=[
            pltpu.VMEM((_RC, D), jnp.float32),
            pltpu.VMEM((_RC, D), jnp.float32),
            pltpu.VMEM((_RC,), jnp.float32),
            pltpu.VMEM((_RC,), jnp.float32),
            pltpu.SemaphoreType.DMA,
            pltpu.SemaphoreType.DMA,
            pltpu.SemaphoreType.DMA,
            pltpu.SemaphoreType.DMA,
        ],
    )
    def k(tab_h, out_h, r0, r1, o0, o1, si0, si1, so0, so1):
        wid = lax.axis_index("s") * _NC + lax.axis_index("c")
        row0 = wid * nch * _RC
        lanes = lax.iota(jnp.int32, _LANES)
        idx15 = lanes * 0 + (_LANES - 1)
        rbufs = (r0, r1)
        obufs = (o0, o1)
        sins = (si0, si1)
        souts = (so0, so1)

        def fire(c, p):
            return pltpu.async_copy(
                tab_h.at[pl.ds(row_off + row0 + c * _RC, _RC)],
                rbufs[p], sins[p])

        def wait_in(c, p):
            pltpu.make_async_copy(
                tab_h.at[pl.ds(row_off + row0 + c * _RC, _RC)],
                rbufs[p], sins[p]).wait()

        def drain_out(p):
            pltpu.make_async_copy(obufs[p], out_h.at[pl.ds(row0, _RC)],
                                  souts[p]).wait()

        def process(c, p):
            rbuf = rbufs[p]
            obuf = obufs[p]

            def group(g, carry):
                acc = jnp.zeros((_LANES,), jnp.float32)
                for j in range(_LANES):
                    r = g * _LANES + j
                    v = rbuf[r, pl.ds(0, _LANES)]
                    for kk in range(1, D // _LANES):
                        v = v + rbuf[r, pl.ds(kk * _LANES, _LANES)]
                    cum = plsc.cumsum(v)
                    tot = lax.gather(
                        cum, idx15.reshape(_LANES, 1),
                        dimension_numbers=lax.GatherDimensionNumbers(
                            offset_dims=(), collapsed_slice_dims=(0,),
                            start_index_map=(0,)),
                        slice_sizes=(1,),
                        mode=lax.GatherScatterMode.PROMISE_IN_BOUNDS)
                    acc = jnp.where(lanes == j, tot, acc)
                obuf[pl.ds(g * _LANES, _LANES)] = acc
                return carry

            lax.fori_loop(0, _RC // _LANES, group, 0)
            pltpu.async_copy(obuf, out_h.at[pl.ds(row0 + c * _RC, _RC)],
                             souts[p])

        fire(0, 0)
        fire(1, 1)

        def step(i, carry):
            for par in (0, 1):
                c = 2 * i + par
                wait_in(c, par)

                @pl.when(c >= 2)
                def _():
                    drain_out(par)

                process(c, par)

                @pl.when(c + 2 < nch)
                def _():
                    fire(c + 2, par)

            return carry

        lax.fori_loop(0, nch // 2, step, 0)
        drain_out(0)
        drain_out(1)

    return k(table)


def _gather_one(idx, tab):
    bl = idx.shape[0]
    tab_n = tab.shape[0]
    per_w = bl // _NW
    nch = per_w // _CHUNK
    mesh = plsc.VectorSubcoreMesh(core_axis_name="c", subcore_axis_name="s")

    @functools.partial(
        pl.kernel,
        mesh=mesh,
        out_type=jax.ShapeDtypeStruct((bl,), jnp.float32),
        scratch_types=[
            pltpu.VMEM((_CHUNK,), jnp.int32),
            pltpu.VMEM((_CHUNK,), jnp.int32),
            pltpu.VMEM((_CHUNK,), jnp.float32),
            pltpu.VMEM((_CHUNK,), jnp.float32),
            pltpu.VMEM_SHARED((tab_n,), jnp.float32),
            pltpu.SemaphoreType.DMA,
            pltpu.SemaphoreType.DMA,
        ],
    )
    def k(idx_h, tab_h, out_h, i0, i1, v0, v1, tab_sh, sem0, sem1):
        wid = lax.axis_index("s") * _NC + lax.axis_index("c")
        idx_bufs = (i0, i1)
        val_bufs = (v0, v1)
        sems = (sem0, sem1)

        def load_idx(c, p):
            base = wid * per_w + c * _CHUNK
            pltpu.sync_copy(idx_h.at[pl.ds(base, _CHUNK)], idx_bufs[p])

        def fire(p):
            return pltpu.async_copy(tab_sh.at[idx_bufs[p]], val_bufs[p],
                                    sems[p])

        pending = {}
        load_idx(0, 0)

        @pl.when(lax.axis_index("s") == 0)
        def _stage():
            pltpu.sync_copy(tab_h, tab_sh)

        plsc.subcore_barrier()
        pending[0] = fire(0)
        for c in range(nch):
            p = c % 2
            if c + 1 < nch:
                load_idx(c + 1, 1 - p)
                pending[c + 1] = fire(1 - p)
            pending.pop(c).wait()
            base = wid * per_w + c * _CHUNK
            pltpu.sync_copy(val_bufs[p], out_h.at[pl.ds(base, _CHUNK)])

    return k(idx, tab)


def _mlp_body(xw_ref, x2_ref, x3_ref, w1_ref, b1_ref, w2_ref, b2_ref, o_ref):
    x = xw_ref[...] + x2_ref[...] + x3_ref[...]
    w1 = w1_ref[...] * SCALE
    h = lax.dot(x, w1, precision=lax.Precision.DEFAULT,
                preferred_element_type=jnp.float32)
    h = jnp.maximum(h + b1_ref[...], 0.0)
    o_ref[...] = lax.dot(h, w2_ref[...], precision=lax.Precision.DEFAULT,
                         preferred_element_type=jnp.float32) + b2_ref[...]


def _mlp(xw, x2, x3, w1, b1, w2, b2, blk_b=1024):
    b, l = xw.shape
    n = w2.shape[1]
    xspec = pl.BlockSpec((blk_b, l), lambda i: (i, 0))
    return pl.pallas_call(
        _mlp_body,
        grid=(b // blk_b,),
        in_specs=[
            xspec,
            xspec,
            xspec,
            pl.BlockSpec((l, D), lambda i: (0, 0)),
            pl.BlockSpec((1, D), lambda i: (0, 0)),
            pl.BlockSpec((D, n), lambda i: (0, 0)),
            pl.BlockSpec((1, n), lambda i: (0, 0)),
        ],
        out_specs=pl.BlockSpec((blk_b, n), lambda i: (i, 0)),
        out_shape=jax.ShapeDtypeStruct((b, n), jnp.float32),
    )(xw, x2, x3, w1, b1, w2, b2)


def kernel(input_ids, input_ids_gram2, input_ids_gram3, input_mask, labels,
           emb_word, emb_g2, emb_g3, W1, b1, W2, b2):
    b, l = input_ids.shape
    v3 = emb_g3.shape[0]
    s3_main = _sc_row_sums(emb_g3, 0, _NCH_RS)
    s2b = _sc_row_sums(emb_g2, _G2_SPLIT, 8)
    s3_tail = _row_sums_block(emb_g3, _SC_ROWS // 8192)[: v3 - _SC_ROWS]
    s2_tail = _row_sums_block(emb_g2, _SC_ROWS // 8192)[: v3 - _SC_ROWS]
    s2a = _row_sums_padded(emb_g2, nblocks=_G2_SPLIT // 8192)
    sw = _row_sums_padded(emb_word)
    xw = _gather_one(input_ids.reshape(-1), sw)
    s2 = jnp.concatenate([s2a, s2b, s2_tail])
    x2 = _gather_one(input_ids_gram2.reshape(-1), s2)
    s3 = jnp.concatenate([s3_main, s3_tail])
    x3 = _gather_one(input_ids_gram3.reshape(-1), s3)
    return _mlp(xw.reshape(b, l), x2.reshape(b, l), x3.reshape(b, l),
                W1, b1.reshape(1, -1), W2, b2.reshape(1, -1))
